# baseline (speedup 1.0000x reference)
import functools

import jax
import jax.numpy as jnp
from jax import lax
from jax.experimental import pallas as pl
from jax.experimental.pallas import tpu as pltpu
from jax.experimental.pallas import tpu_sc as plsc

B = 4096
VOCAB = 100000
WIDE_DIM = 8
DEEP_DIM = 26
N_WIDE = 2
WD = 16
N_DEEP = 6
DD = 32
DEEP_RAW = DEEP_DIM - N_DEEP
WIDE_RAW = WIDE_DIM - N_WIDE
H0, H1 = 256, 128
WIDE_OUT = N_WIDE * WD + WIDE_RAW
ACTION_DIM = 64

NC, NS = 2, 16
NW = NC * NS
LANES = 16

_SC_PARAMS = pltpu.CompilerParams(
    use_tc_tiling_on_sc=True, needs_layout_passes=False)


def _gather_one_row(tab, out, idx_src, d, row_v, idx_v, out_v, sem):
    cp = pltpu.async_copy(tab.at[d], row_v, sem)
    pltpu.sync_copy(idx_src, idx_v)
    cp.wait()

    def gath(i, _):
        sl = pl.ds(i * LANES, LANES)
        out_v[sl] = plsc.load_gather(row_v, [idx_v[sl]])
        return 0

    lax.fori_loop(0, B // LANES, gath, 0)
    pltpu.sync_copy(out_v, out.at[d])


@functools.cache
def _build_deep_gather():
    mesh = plsc.VectorSubcoreMesh(
        core_axis_name="c", subcore_axis_name="s", num_cores=NC, num_subcores=NS
    )
    out_type = [jax.ShapeDtypeStruct((DD, B), jnp.float32)] * N_DEEP
    scratch = [
        pltpu.VMEM((VOCAB,), jnp.float32),
        pltpu.VMEM((B,), jnp.int32),
        pltpu.VMEM((B,), jnp.float32),
        pltpu.SemaphoreType.DMA,
    ]

    @functools.partial(
        pl.kernel, mesh=mesh, out_type=out_type, scratch_types=scratch,
        compiler_params=_SC_PARAMS)
    def deep_k(idxd, edt0, edt1, edt2, edt3, edt4, edt5,
               gd0, gd1, gd2, gd3, gd4, gd5,
               row_v, idx_v, out_v, sem):
        wid = lax.axis_index("s") * NC + lax.axis_index("c")
        tabs = [edt0, edt1, edt2, edt3, edt4, edt5]
        outs = [gd0, gd1, gd2, gd3, gd4, gd5]
        for k in range(N_DEEP):
            _gather_one_row(tabs[k], outs[k], idxd.at[k], wid,
                            row_v, idx_v, out_v, sem)

    return deep_k


@functools.cache
def _build_wide_gather():
    mesh = plsc.VectorSubcoreMesh(
        core_axis_name="c", subcore_axis_name="s", num_cores=NC, num_subcores=NS
    )
    out_type = [jax.ShapeDtypeStruct((WD, B), jnp.float32)] * N_WIDE
    scratch = [
        pltpu.VMEM((VOCAB,), jnp.float32),
        pltpu.VMEM((B,), jnp.int32),
        pltpu.VMEM((B,), jnp.float32),
        pltpu.SemaphoreType.DMA,
    ]

    @functools.partial(
        pl.kernel, mesh=mesh, out_type=out_type, scratch_types=scratch,
        compiler_params=_SC_PARAMS)
    def wide_k(idxw, ewt0, ewt1, gw0, gw1, row_v, idx_v, out_v, sem):
        wid = lax.axis_index("s") * NC + lax.axis_index("c")
        tabs = [ewt0, ewt1]
        outs = [gw0, gw1]
        for t in range(N_WIDE):
            @pl.when((wid >= t * WD) & (wid < (t + 1) * WD))
            def _(t=t):
                _gather_one_row(tabs[t], outs[t], idxw.at[t], wid - t * WD,
                                row_v, idx_v, out_v, sem)

    return wide_k


BLK = 1024
GRID = B // BLK


def _col_spec(d):
    return pl.BlockSpec((d, BLK), lambda i: (0, i))


def _full_spec(shape):
    return pl.BlockSpec(shape, lambda i: (0,) * len(shape))


def _deep_mlp_body(xt, gd0, gd1, gd2, gd3, gd4, gd5, w0, b0, w1, b1, out):
    dt = jnp.concatenate(
        [gd0[...], gd1[...], gd2[...], gd3[...], gd4[...], gd5[...],
         xt[WIDE_DIM + N_DEEP:, :]], axis=0)
    h = jnp.dot(w0[...], dt, preferred_element_type=jnp.float32) + b0[...]
    h = jnp.maximum(h, 0.0)
    h = jnp.dot(w1[...], h, preferred_element_type=jnp.float32) + b1[...]
    out[...] = jnp.maximum(h, 0.0)


def _deep_mlp(xt, gds, w0, b0, w1, b1, interpret=False):
    in_specs = (
        [_col_spec(WIDE_DIM + DEEP_DIM)]
        + [_col_spec(DD)] * N_DEEP
        + [_full_spec(w0.shape), _full_spec(b0.shape),
           _full_spec(w1.shape), _full_spec(b1.shape)]
    )
    return pl.pallas_call(
        _deep_mlp_body,
        grid=(GRID,),
        in_specs=in_specs,
        out_specs=_col_spec(H1),
        out_shape=jax.ShapeDtypeStruct((H1, B), jnp.float32),
        interpret=interpret,
    )(xt, *gds, w0, b0, w1, b1)


def _head_body(xt, gw0, gw1, h1, wl, bl, out):
    zt = jnp.concatenate(
        [gw0[...], gw1[...], xt[N_WIDE:WIDE_DIM, :], h1[...]], axis=0)
    out[...] = jnp.dot(wl[...], zt, preferred_element_type=jnp.float32) + bl[...]


def _head(xt, gws, h1, wl, bl, interpret=False):
    in_specs = (
        [_col_spec(WIDE_DIM + DEEP_DIM)]
        + [_col_spec(WD)] * N_WIDE
        + [_col_spec(H1), _full_spec(wl.shape), _full_spec(bl.shape)]
    )
    return pl.pallas_call(
        _head_body,
        grid=(GRID,),
        in_specs=in_specs,
        out_specs=_col_spec(ACTION_DIM),
        out_shape=jax.ShapeDtypeStruct((ACTION_DIM, B), jnp.float32),
        interpret=interpret,
    )(xt, *gws, h1, wl, bl)


def kernel(x, Ew0, Ew1, Ed0, Ed1, Ed2, Ed3, Ed4, Ed5, W0, b0, W1, b1, Wl, bl):
    xt = x.T
    idxd = xt[WIDE_DIM:WIDE_DIM + N_DEEP, :].astype(jnp.int32)
    idxw = xt[0:N_WIDE, :].astype(jnp.int32)
    gds = _build_deep_gather()(idxd, Ed0.T, Ed1.T, Ed2.T, Ed3.T, Ed4.T, Ed5.T)
    gws = _build_wide_gather()(idxw, Ew0.T, Ew1.T)
    h1 = _deep_mlp(xt, gds, W0, b0[:, None], W1, b1[:, None])
    out_t = _head(xt, gws, h1, Wl, bl[:, None])
    return out_t.T


# speedup vs baseline: 1.1658x; 1.1658x over previous
import functools

import jax
import jax.numpy as jnp
from jax import lax
from jax.experimental import pallas as pl
from jax.experimental.pallas import tpu as pltpu
from jax.experimental.pallas import tpu_sc as plsc

B = 4096
VOCAB = 100000
WIDE_DIM = 8
DEEP_DIM = 26
N_WIDE = 2
WD = 16
N_DEEP = 6
DD = 32
N_TAB = N_WIDE + N_DEEP
DEEP_RAW = DEEP_DIM - N_DEEP
WIDE_RAW = WIDE_DIM - N_WIDE
H0, H1 = 256, 128
WIDE_OUT = N_WIDE * WD + WIDE_RAW
ACTION_DIM = 64

NC, NS = 2, 16
NW = NC * NS
LANES = 16

TOTAL_ROWS = N_WIDE * WD + N_DEEP * DD
RPW = TOTAL_ROWS // NW
FR = [0, WD, 2 * WD] + [2 * WD + (i + 1) * DD for i in range(N_DEEP)]
COLS = [0, 1] + [WIDE_DIM + i for i in range(N_DEEP)]


@functools.cache
def _build_gather():
    mesh = plsc.VectorSubcoreMesh(
        core_axis_name="c", subcore_axis_name="s", num_cores=NC, num_subcores=NS
    )
    out_type = (
        [jax.ShapeDtypeStruct((WD, B), jnp.float32)] * N_WIDE
        + [jax.ShapeDtypeStruct((DD, B), jnp.float32)] * N_DEEP
    )
    scratch = [
        pltpu.VMEM((VOCAB,), jnp.float32),
        pltpu.VMEM((B,), jnp.float32),
        pltpu.VMEM((B,), jnp.int32),
        pltpu.VMEM((B,), jnp.float32),
        pltpu.SemaphoreType.DMA,
    ]

    @functools.partial(
        pl.kernel, mesh=mesh, out_type=out_type, scratch_types=scratch,
        compiler_params=pltpu.CompilerParams(
            use_tc_tiling_on_sc=True, needs_layout_passes=False))
    def gather_k(xt, ewt0, ewt1, edt0, edt1, edt2, edt3, edt4, edt5,
                 gw0, gw1, gd0, gd1, gd2, gd3, gd4, gd5,
                 row_v, xf_v, idx_v, out_v, sem):
        wid = lax.axis_index("s") * NC + lax.axis_index("c")
        start = wid * RPW
        tabs = [ewt0, ewt1, edt0, edt1, edt2, edt3, edt4, edt5]
        outs = [gw0, gw1, gd0, gd1, gd2, gd3, gd4, gd5]

        for t in range(N_TAB):
            lo, hi = FR[t], FR[t + 1]

            @pl.when((start < hi) & (start + RPW > lo))
            def _(t=t, lo=lo, hi=hi):
                tab, out = tabs[t], outs[t]
                pltpu.sync_copy(xt.at[COLS[t]], xf_v)

                def conv(i, _):
                    s0 = pl.ds(i * 2 * LANES, LANES)
                    s1 = pl.ds(i * 2 * LANES + LANES, LANES)
                    idx_v[s0] = xf_v[s0].astype(jnp.int32)
                    idx_v[s1] = xf_v[s1].astype(jnp.int32)
                    return 0

                lax.fori_loop(0, B // (2 * LANES), conv, 0)
                k_lo = jnp.maximum(lo - start, 0)
                k_hi = jnp.minimum(hi - start, RPW)

                def row_body(k, _):
                    d = start + k - lo
                    pltpu.async_copy(tab.at[d], row_v, sem).wait()

                    def gath(i, _):
                        s0 = pl.ds(i * 2 * LANES, LANES)
                        s1 = pl.ds(i * 2 * LANES + LANES, LANES)
                        out_v[s0] = plsc.load_gather(row_v, [idx_v[s0]])
                        out_v[s1] = plsc.load_gather(row_v, [idx_v[s1]])
                        return 0

                    lax.fori_loop(0, B // (2 * LANES), gath, 0)
                    pltpu.sync_copy(out_v, out.at[d])
                    return 0

                lax.fori_loop(k_lo, k_hi, row_body, 0)

    return gather_k


BLK = 1024
GRID = B // BLK


def _mlp_body(xt, gw0, gw1, gd0, gd1, gd2, gd3, gd4, gd5,
              w0, b0, w1, b1, wl, bl, out):
    dt = jnp.concatenate(
        [gd0[...], gd1[...], gd2[...], gd3[...], gd4[...], gd5[...],
         xt[WIDE_DIM + N_DEEP:, :]], axis=0)
    h = jnp.dot(w0[...], dt, preferred_element_type=jnp.float32) + b0[...]
    h = jnp.maximum(h, 0.0)
    h = jnp.dot(w1[...], h, preferred_element_type=jnp.float32) + b1[...]
    h = jnp.maximum(h, 0.0)
    wt = jnp.concatenate(
        [gw0[...], gw1[...], xt[N_WIDE:WIDE_DIM, :]], axis=0)
    zt = jnp.concatenate([wt, h], axis=0)
    out[...] = jnp.dot(wl[...], zt, preferred_element_type=jnp.float32) + bl[...]


def _col_spec(d):
    return pl.BlockSpec((d, BLK), lambda i: (0, i))


def _full_spec(shape):
    return pl.BlockSpec(shape, lambda i: (0,) * len(shape))


def _mlp(xt, gws, gds, w0, b0, w1, b1, wl, bl, interpret=False):
    in_specs = (
        [_col_spec(WIDE_DIM + DEEP_DIM)]
        + [_col_spec(WD)] * N_WIDE
        + [_col_spec(DD)] * N_DEEP
        + [_full_spec(w0.shape), _full_spec(b0.shape), _full_spec(w1.shape),
           _full_spec(b1.shape), _full_spec(wl.shape), _full_spec(bl.shape)]
    )
    return pl.pallas_call(
        _mlp_body,
        grid=(GRID,),
        in_specs=in_specs,
        out_specs=_col_spec(ACTION_DIM),
        out_shape=jax.ShapeDtypeStruct((ACTION_DIM, B), jnp.float32),
        interpret=interpret,
    )(xt, *gws, *gds, w0, b0, w1, b1, wl, bl)


def kernel(x, Ew0, Ew1, Ed0, Ed1, Ed2, Ed3, Ed4, Ed5, W0, b0, W1, b1, Wl, bl):
    xt = x.T
    gathered = _build_gather()(
        xt, Ew0.T, Ew1.T, Ed0.T, Ed1.T, Ed2.T, Ed3.T, Ed4.T, Ed5.T)
    gws = gathered[:N_WIDE]
    gds = gathered[N_WIDE:]
    out_t = _mlp(xt, gws, gds,
                 W0, b0[:, None], W1, b1[:, None], Wl, bl[:, None])
    return out_t.T


# speedup vs baseline: 1.1931x; 1.0235x over previous
import functools

import jax
import jax.numpy as jnp
from jax import lax
from jax.experimental import pallas as pl
from jax.experimental.pallas import tpu as pltpu
from jax.experimental.pallas import tpu_sc as plsc

B = 4096
VOCAB = 100000
WIDE_DIM = 8
DEEP_DIM = 26
N_WIDE = 2
WD = 16
N_DEEP = 6
DD = 32
N_TAB = N_WIDE + N_DEEP
DEEP_RAW = DEEP_DIM - N_DEEP
WIDE_RAW = WIDE_DIM - N_WIDE
H0, H1 = 256, 128
WIDE_OUT = N_WIDE * WD + WIDE_RAW
ACTION_DIM = 64

NC, NS = 2, 16
NW = NC * NS
LANES = 16

TOTAL_ROWS = N_WIDE * WD + N_DEEP * DD
RPW = TOTAL_ROWS // NW
FR = [0, WD, 2 * WD] + [2 * WD + (i + 1) * DD for i in range(N_DEEP)]
COLS = [0, 1] + [WIDE_DIM + i for i in range(N_DEEP)]


@functools.cache
def _build_gather():
    mesh = plsc.VectorSubcoreMesh(
        core_axis_name="c", subcore_axis_name="s", num_cores=NC, num_subcores=NS
    )
    out_type = (
        [jax.ShapeDtypeStruct((WD, B), jnp.float32)] * N_WIDE
        + [jax.ShapeDtypeStruct((DD, B), jnp.float32)] * N_DEEP
    )
    scratch = [
        pltpu.VMEM((VOCAB,), jnp.float32),
        pltpu.VMEM((B,), jnp.float32),
        pltpu.VMEM((B,), jnp.int32),
        pltpu.VMEM((B,), jnp.float32),
        pltpu.SemaphoreType.DMA,
    ]

    @functools.partial(
        pl.kernel, mesh=mesh, out_type=out_type, scratch_types=scratch,
        compiler_params=pltpu.CompilerParams(
            use_tc_tiling_on_sc=True, needs_layout_passes=False,
            skip_device_barrier=True))
    def gather_k(xt, ewt0, ewt1, edt0, edt1, edt2, edt3, edt4, edt5,
                 gw0, gw1, gd0, gd1, gd2, gd3, gd4, gd5,
                 row_v, xf_v, idx_v, out_v, sem):
        wid = lax.axis_index("s") * NC + lax.axis_index("c")
        start = wid * RPW
        tabs = [ewt0, ewt1, edt0, edt1, edt2, edt3, edt4, edt5]
        outs = [gw0, gw1, gd0, gd1, gd2, gd3, gd4, gd5]

        for t in range(N_TAB):
            lo, hi = FR[t], FR[t + 1]

            @pl.when((start < hi) & (start + RPW > lo))
            def _(t=t, lo=lo, hi=hi):
                tab, out = tabs[t], outs[t]
                pltpu.sync_copy(xt.at[COLS[t]], xf_v)

                def conv(i, _):
                    s0 = pl.ds(i * 2 * LANES, LANES)
                    s1 = pl.ds(i * 2 * LANES + LANES, LANES)
                    idx_v[s0] = xf_v[s0].astype(jnp.int32)
                    idx_v[s1] = xf_v[s1].astype(jnp.int32)
                    return 0

                lax.fori_loop(0, B // (2 * LANES), conv, 0)
                k_lo = jnp.maximum(lo - start, 0)
                k_hi = jnp.minimum(hi - start, RPW)

                def row_body(k, _):
                    d = start + k - lo
                    pltpu.async_copy(tab.at[d], row_v, sem).wait()

                    def gath(i, _):
                        s0 = pl.ds(i * 2 * LANES, LANES)
                        s1 = pl.ds(i * 2 * LANES + LANES, LANES)
                        out_v[s0] = plsc.load_gather(row_v, [idx_v[s0]])
                        out_v[s1] = plsc.load_gather(row_v, [idx_v[s1]])
                        return 0

                    lax.fori_loop(0, B // (2 * LANES), gath, 0)
                    pltpu.sync_copy(out_v, out.at[d])
                    return 0

                lax.fori_loop(k_lo, k_hi, row_body, 0)

    return gather_k


BLK = 2048
GRID = B // BLK


def _mlp_body(xt, gw0, gw1, gd0, gd1, gd2, gd3, gd4, gd5,
              w0, b0, w1, b1, wl, bl, out):
    dt = jnp.concatenate(
        [gd0[...], gd1[...], gd2[...], gd3[...], gd4[...], gd5[...],
         xt[WIDE_DIM + N_DEEP:, :]], axis=0)
    h = jnp.dot(w0[...], dt, preferred_element_type=jnp.float32) + b0[...]
    h = jnp.maximum(h, 0.0)
    h = jnp.dot(w1[...], h, preferred_element_type=jnp.float32) + b1[...]
    h = jnp.maximum(h, 0.0)
    wt = jnp.concatenate(
        [gw0[...], gw1[...], xt[N_WIDE:WIDE_DIM, :]], axis=0)
    zt = jnp.concatenate([wt, h], axis=0)
    out[...] = jnp.dot(wl[...], zt, preferred_element_type=jnp.float32) + bl[...]


def _col_spec(d):
    return pl.BlockSpec((d, BLK), lambda i: (0, i))


def _full_spec(shape):
    return pl.BlockSpec(shape, lambda i: (0,) * len(shape))


def _mlp(xt, gws, gds, w0, b0, w1, b1, wl, bl, interpret=False):
    in_specs = (
        [_col_spec(WIDE_DIM + DEEP_DIM)]
        + [_col_spec(WD)] * N_WIDE
        + [_col_spec(DD)] * N_DEEP
        + [_full_spec(w0.shape), _full_spec(b0.shape), _full_spec(w1.shape),
           _full_spec(b1.shape), _full_spec(wl.shape), _full_spec(bl.shape)]
    )
    return pl.pallas_call(
        _mlp_body,
        grid=(GRID,),
        in_specs=in_specs,
        out_specs=_col_spec(ACTION_DIM),
        out_shape=jax.ShapeDtypeStruct((ACTION_DIM, B), jnp.float32),
        interpret=interpret,
    )(xt, *gws, *gds, w0, b0, w1, b1, wl, bl)


def kernel(x, Ew0, Ew1, Ed0, Ed1, Ed2, Ed3, Ed4, Ed5, W0, b0, W1, b1, Wl, bl):
    xt = x.T
    gathered = _build_gather()(
        xt, Ew0.T, Ew1.T, Ed0.T, Ed1.T, Ed2.T, Ed3.T, Ed4.T, Ed5.T)
    gws = gathered[:N_WIDE]
    gds = gathered[N_WIDE:]
    out_t = _mlp(xt, gws, gds,
                 W0, b0[:, None], W1, b1[:, None], Wl, bl[:, None])
    return out_t.T


# speedup vs baseline: 1.2095x; 1.0137x over previous
import functools

import jax
import jax.numpy as jnp
from jax import lax
from jax.experimental import pallas as pl
from jax.experimental.pallas import tpu as pltpu
from jax.experimental.pallas import tpu_sc as plsc

B = 4096
VOCAB = 100000
WIDE_DIM = 8
DEEP_DIM = 26
N_WIDE = 2
WD = 16
N_DEEP = 6
DD = 32
N_TAB = N_WIDE + N_DEEP
DEEP_RAW = DEEP_DIM - N_DEEP
WIDE_RAW = WIDE_DIM - N_WIDE
H0, H1 = 256, 128
WIDE_OUT = N_WIDE * WD + WIDE_RAW
ACTION_DIM = 64

NC, NS = 2, 16
NW = NC * NS
LANES = 16

TOTAL_ROWS = N_WIDE * WD + N_DEEP * DD
RPW = TOTAL_ROWS // NW
FR = [0, WD, 2 * WD] + [2 * WD + (i + 1) * DD for i in range(N_DEEP)]
COLS = [0, 1] + [WIDE_DIM + i for i in range(N_DEEP)]


@functools.cache
def _build_gather():
    mesh = plsc.VectorSubcoreMesh(
        core_axis_name="c", subcore_axis_name="s", num_cores=NC, num_subcores=NS
    )
    out_type = (
        [jax.ShapeDtypeStruct((WD, B), jnp.float32)] * N_WIDE
        + [jax.ShapeDtypeStruct((DD, B), jnp.float32)] * N_DEEP
    )
    scratch = [
        pltpu.VMEM((VOCAB,), jnp.float32),
        pltpu.VMEM((B,), jnp.int32),
        pltpu.VMEM((B,), jnp.float32),
        pltpu.SemaphoreType.DMA,
    ]

    @functools.partial(
        pl.kernel, mesh=mesh, out_type=out_type, scratch_types=scratch,
        compiler_params=pltpu.CompilerParams(
            use_tc_tiling_on_sc=True, needs_layout_passes=False,
            skip_device_barrier=True))
    def gather_k(idx8, ewt0, ewt1, edt0, edt1, edt2, edt3, edt4, edt5,
                 gw0, gw1, gd0, gd1, gd2, gd3, gd4, gd5,
                 row_v, idx_v, out_v, sem):
        wid = lax.axis_index("s") * NC + lax.axis_index("c")
        start = wid * RPW
        tabs = [ewt0, ewt1, edt0, edt1, edt2, edt3, edt4, edt5]
        outs = [gw0, gw1, gd0, gd1, gd2, gd3, gd4, gd5]

        for t in range(N_TAB):
            lo, hi = FR[t], FR[t + 1]

            @pl.when((start < hi) & (start + RPW > lo))
            def _(t=t, lo=lo, hi=hi):
                tab, out = tabs[t], outs[t]
                pltpu.sync_copy(idx8.at[t], idx_v)
                k_lo = jnp.maximum(lo - start, 0)
                k_hi = jnp.minimum(hi - start, RPW)

                def row_body(k, _):
                    d = start + k - lo
                    pltpu.async_copy(tab.at[d], row_v, sem).wait()

                    def gath(i, _):
                        s0 = pl.ds(i * 2 * LANES, LANES)
                        s1 = pl.ds(i * 2 * LANES + LANES, LANES)
                        out_v[s0] = plsc.load_gather(row_v, [idx_v[s0]])
                        out_v[s1] = plsc.load_gather(row_v, [idx_v[s1]])
                        return 0

                    lax.fori_loop(0, B // (2 * LANES), gath, 0)
                    pltpu.sync_copy(out_v, out.at[d])
                    return 0

                lax.fori_loop(k_lo, k_hi, row_body, 0)

    return gather_k


BLK = 2048
GRID = B // BLK


def _mlp_body(xt, gw0, gw1, gd0, gd1, gd2, gd3, gd4, gd5,
              w0, b0, w1, b1, wl, bl, out):
    dt = jnp.concatenate(
        [gd0[...], gd1[...], gd2[...], gd3[...], gd4[...], gd5[...],
         xt[WIDE_DIM + N_DEEP:, :]], axis=0)
    h = jnp.dot(w0[...], dt, preferred_element_type=jnp.float32) + b0[...]
    h = jnp.maximum(h, 0.0)
    h = jnp.dot(w1[...], h, preferred_element_type=jnp.float32) + b1[...]
    h = jnp.maximum(h, 0.0)
    wt = jnp.concatenate(
        [gw0[...], gw1[...], xt[N_WIDE:WIDE_DIM, :]], axis=0)
    zt = jnp.concatenate([wt, h], axis=0)
    out[...] = jnp.dot(wl[...], zt, preferred_element_type=jnp.float32) + bl[...]


def _col_spec(d):
    return pl.BlockSpec((d, BLK), lambda i: (0, i))


def _full_spec(shape):
    return pl.BlockSpec(shape, lambda i: (0,) * len(shape))


def _mlp(xt, gws, gds, w0, b0, w1, b1, wl, bl, interpret=False):
    in_specs = (
        [_col_spec(WIDE_DIM + DEEP_DIM)]
        + [_col_spec(WD)] * N_WIDE
        + [_col_spec(DD)] * N_DEEP
        + [_full_spec(w0.shape), _full_spec(b0.shape), _full_spec(w1.shape),
           _full_spec(b1.shape), _full_spec(wl.shape), _full_spec(bl.shape)]
    )
    return pl.pallas_call(
        _mlp_body,
        grid=(GRID,),
        in_specs=in_specs,
        out_specs=_col_spec(ACTION_DIM),
        out_shape=jax.ShapeDtypeStruct((ACTION_DIM, B), jnp.float32),
        interpret=interpret,
    )(xt, *gws, *gds, w0, b0, w1, b1, wl, bl)


def kernel(x, Ew0, Ew1, Ed0, Ed1, Ed2, Ed3, Ed4, Ed5, W0, b0, W1, b1, Wl, bl):
    xt = x.T
    idx8 = jnp.concatenate(
        [xt[0:N_WIDE, :], xt[WIDE_DIM:WIDE_DIM + N_DEEP, :]], axis=0
    ).astype(jnp.int32)
    gathered = _build_gather()(
        idx8, Ew0.T, Ew1.T, Ed0.T, Ed1.T, Ed2.T, Ed3.T, Ed4.T, Ed5.T)
    gws = gathered[:N_WIDE]
    gds = gathered[N_WIDE:]
    out_t = _mlp(xt, gws, gds,
                 W0, b0[:, None], W1, b1[:, None], Wl, bl[:, None])
    return out_t.T
